# Initial kernel scaffold; baseline (speedup 1.0000x reference)
import jax
import jax.numpy as jnp
from jax.experimental import pallas as pl


def kernel(latent_z, relation, gamma, sample_idx, epoch):
    raise NotImplementedError("write your pallas kernel here")


# speedup vs baseline: 1.0585x; 1.0585x over previous
import functools

import jax
import jax.numpy as jnp
from jax import lax
from jax.experimental import pallas as pl
from jax.experimental.pallas import tpu as pltpu
from jax.experimental.pallas import tpu_sc as plsc

_NC = 2
_NS = 16
_NW = _NC * _NS
_LANES = 16


def _sc_gather(relation, sample_idx, latent_z):
  n = relation.shape[0]
  s = sample_idx.shape[0]
  d = latent_z.shape[1]
  rows_per_w = s // _NW
  chunk = 8
  nchunk = rows_per_w // chunk

  mesh = plsc.VectorSubcoreMesh(core_axis_name="c", subcore_axis_name="s")

  @functools.partial(
      pl.kernel,
      out_type=(
          jax.ShapeDtypeStruct((s, s), jnp.float32),
          jax.ShapeDtypeStruct((s, d), jnp.float32),
      ),
      mesh=mesh,
      scratch_types=[
          pltpu.VMEM((s,), jnp.int32),
          pltpu.VMEM((chunk, n), jnp.float32),
          pltpu.VMEM((chunk, s), jnp.float32),
          pltpu.VMEM((rows_per_w, d), jnp.float32),
          pltpu.SemaphoreType.DMA,
          pltpu.SemaphoreType.DMA,
      ],
      compiler_params=pltpu.CompilerParams(use_tc_tiling_on_sc=False),
  )
  def k(rel_hbm, idx_hbm, z_hbm, r_hbm, zs_hbm,
        idx_v, rows_v, out_v, zs_v, sem_rows, sem_z):
    wid = lax.axis_index("s") * _NC + lax.axis_index("c")
    base = wid * rows_per_w

    pltpu.sync_copy(idx_hbm, idx_v)

    z_cp = pltpu.async_copy(z_hbm.at[idx_v.at[pl.ds(base, rows_per_w)]],
                            zs_v, sem_z)

    for c in range(nchunk):
      row0 = base + c * chunk
      pltpu.async_copy(
          rel_hbm.at[idx_v.at[pl.ds(row0, chunk)]], rows_v, sem_rows
      ).wait()
      for r in range(chunk):
        row_ids = jnp.full((_LANES,), r, jnp.int32)

        @functools.partial(plsc.parallel_loop, 0, s // _LANES, unroll=4)
        def _(kk, _r=r, _row_ids=row_ids):
          cols = idx_v[pl.ds(kk * _LANES, _LANES)]
          vals = plsc.load_gather(rows_v, [_row_ids, cols])
          out_v[_r, pl.ds(kk * _LANES, _LANES)] = vals

      pltpu.sync_copy(out_v, r_hbm.at[pl.ds(row0, chunk)])

    z_cp.wait()
    pltpu.sync_copy(zs_v, zs_hbm.at[pl.ds(base, rows_per_w)])

  return k(relation, sample_idx, latent_z)


def _tc_loss(r_mat, zs):
  s, d = zs.shape
  bm = 256
  grid = s // bm

  def body(r_ref, zs_ref, out_ref):
    i = pl.program_id(0)
    zall = zs_ref[...]
    zsb = zs_ref[pl.ds(i * bm, bm), :]
    g = lax.dot_general(zsb, zall, (((1,), (1,)), ((), ())),
                        preferred_element_type=jnp.float32)
    nb = jnp.sum(zsb * zsb, axis=1)[:, None]
    nz = jnp.sum(zall * zall, axis=1)[None, :]
    d2 = nb + nz - 2.0 * g
    dm = jnp.where(d2 > 0, jnp.sqrt(jnp.where(d2 > 0, d2, 1.0)), 0.0)
    rows = i * bm + lax.broadcasted_iota(jnp.int32, (bm, s), 0)
    cols = lax.broadcasted_iota(jnp.int32, (bm, s), 1)
    diag = rows == cols
    dm = jnp.where(diag, 0.0, dm)
    rb = r_ref[...]
    rd = jnp.where(diag, 5.0, rb)
    num = dm - rb
    part = jnp.sum(num * num / rd)

    @pl.when(i == 0)
    def _():
      out_ref[0, 0] = 0.0

    out_ref[0, 0] += part

    @pl.when(i == grid - 1)
    def _():
      out_ref[0, 0] = jnp.sqrt(out_ref[0, 0])

  out = pl.pallas_call(
      body,
      grid=(grid,),
      in_specs=[
          pl.BlockSpec((bm, s), lambda i: (i, 0)),
          pl.BlockSpec((s, d), lambda i: (0, 0)),
      ],
      out_specs=pl.BlockSpec(memory_space=pltpu.SMEM),
      out_shape=jax.ShapeDtypeStruct((1, 1), jnp.float32),
  )(r_mat, zs)
  return out[0, 0]


@jax.jit
def kernel(latent_z, relation, gamma, sample_idx, epoch):
  del gamma, epoch
  idx = sample_idx.astype(jnp.int32)
  r_mat, zs = _sc_gather(relation, idx, latent_z)
  return _tc_loss(r_mat, zs)
